# Initial kernel scaffold; baseline (speedup 1.0000x reference)
import jax
import jax.numpy as jnp
from jax.experimental import pallas as pl


def kernel(xyzt, table0, table1, table2, table3, st_table1, W_out, b_out):
    raise NotImplementedError("write your pallas kernel here")


# speedup vs baseline: 1.3191x; 1.3191x over previous
import functools

import jax
import jax.numpy as jnp
from jax import lax
from jax.experimental import pallas as pl
from jax.experimental.pallas import tpu as pltpu
from jax.experimental.pallas import tpu_sc as plsc

_NC = 2
_NS = 16
_NW = _NC * _NS
_LANES = 16

_TABLE_DIMS = ((128, 32), (64, 64), (32, 128), (16, 256), (16, 64))


@functools.lru_cache(maxsize=None)
def _make_gather(B):
    BPW = B // _NW
    CH = 128
    NCH = BPW // CH
    NG = BPW // _LANES

    mesh = plsc.VectorSubcoreMesh(core_axis_name="c", subcore_axis_name="s")

    out_type = [jax.ShapeDtypeStruct((B, d), jnp.float32)
                for (_, d) in _TABLE_DIMS]
    scratch_types = [
        pltpu.VMEM((BPW * 4,), jnp.float32),
        pltpu.VMEM((BPW,), jnp.int32),
        pltpu.VMEM((BPW,), jnp.int32),
        pltpu.VMEM((BPW,), jnp.int32),
        pltpu.VMEM((BPW,), jnp.int32),
        pltpu.VMEM((BPW,), jnp.int32),
        pltpu.VMEM((CH, 32), jnp.float32),
        pltpu.VMEM((CH, 64), jnp.float32),
        pltpu.VMEM((CH, 128), jnp.float32),
        pltpu.VMEM((CH, 256), jnp.float32),
        pltpu.VMEM((CH, 64), jnp.float32),
        pltpu.SemaphoreType.DMA,
    ]

    @functools.partial(
        pl.kernel, mesh=mesh, out_type=out_type, scratch_types=scratch_types,
        compiler_params=pltpu.CompilerParams(use_tc_tiling_on_sc=False))
    def gather_kernel(xyzt_hbm, t0, t1, t2, t3, t4,
                      o0, o1, o2, o3, o4,
                      coords, i0, i1, i2, i3, i4,
                      b0, b1, b2, b3, b4, sem):
        wid = lax.axis_index("s") * _NC + lax.axis_index("c")
        base = wid * BPW
        for c in range(4):
            pltpu.sync_copy(xyzt_hbm.at[pl.ds(c * B + base, BPW)],
                            coords.at[pl.ds(c * BPW, BPW)])

        def body(g, carry):
            off = pl.multiple_of(g * _LANES, _LANES)
            x = coords[pl.ds(off, _LANES)]
            y = coords[pl.ds(BPW + off, _LANES)]
            z = coords[pl.ds(2 * BPW + off, _LANES)]
            t = coords[pl.ds(3 * BPW + off, _LANES)]
            ix = jnp.clip((x * 128.0).astype(jnp.int32), 0, 127)
            iy = jnp.clip((y * 128.0).astype(jnp.int32), 0, 127)
            iz = jnp.clip((z * 128.0).astype(jnp.int32), 0, 127)
            it = jnp.clip((t * 64.0).astype(jnp.int32), 0, 63)
            idx0 = (ix * 128 + iy) * 128 + iz
            idx1 = ((ix >> 1) * 64 + (iy >> 1)) * 64 + (iz >> 1)
            idx2 = ((ix >> 2) * 32 + (iy >> 2)) * 32 + (iz >> 2)
            idx3 = ((ix >> 3) * 16 + (iy >> 3)) * 16 + (iz >> 3)
            idx4 = idx3 * 64 + it
            sl = pl.ds(pl.multiple_of(g * _LANES, _LANES), _LANES)
            i0[sl] = idx0
            i1[sl] = idx1
            i2[sl] = idx2
            i3[sl] = idx3
            i4[sl] = idx4
            return carry

        lax.fori_loop(0, NG, body, 0)

        for tbl, idx, buf, out in ((t0, i0, b0, o0), (t1, i1, b1, o1),
                                   (t2, i2, b2, o2), (t3, i3, b3, o3),
                                   (t4, i4, b4, o4)):
            for j in range(NCH):
                isl = idx.at[pl.ds(j * CH, CH)]
                pltpu.async_copy(tbl.at[isl], buf, sem).wait()
                pltpu.sync_copy(buf, out.at[pl.ds(base + j * CH, CH)])

    return gather_kernel


def _matmul_body(f0, f1, f2, f3, f4, w0, w1, w2, w3, w4, b, out):
    acc = jnp.dot(f0[...], w0[...], preferred_element_type=jnp.float32)
    acc += jnp.dot(f1[...], w1[...], preferred_element_type=jnp.float32)
    acc += jnp.dot(f2[...], w2[...], preferred_element_type=jnp.float32)
    acc += jnp.dot(f3[...], w3[...], preferred_element_type=jnp.float32)
    acc += jnp.dot(f4[...], w4[...], preferred_element_type=jnp.float32)
    out[...] = acc + b[...]


@functools.lru_cache(maxsize=None)
def _make_matmul(B, BM=1024):
    grid = (B // BM,)
    fspec = lambda d: pl.BlockSpec((BM, d), lambda i: (i, 0))
    wspec = lambda d: pl.BlockSpec((d, 4), lambda i: (0, 0))
    return pl.pallas_call(
        _matmul_body,
        grid=grid,
        in_specs=[fspec(d) for (_, d) in _TABLE_DIMS]
        + [wspec(d) for (_, d) in _TABLE_DIMS]
        + [pl.BlockSpec((1, 4), lambda i: (0, 0))],
        out_specs=pl.BlockSpec((BM, 4), lambda i: (i, 0)),
        out_shape=jax.ShapeDtypeStruct((B, 4), jnp.float32),
    )


def kernel(xyzt, table0, table1, table2, table3, st_table1, W_out, b_out):
    B = xyzt.shape[0]
    tables = [table0.reshape(-1, 32), table1.reshape(-1, 64),
              table2.reshape(-1, 128), table3.reshape(-1, 256),
              st_table1.reshape(-1, 64)]
    feats = _make_gather(B)(xyzt.T.reshape(-1), *tables)
    offs = [0, 32, 96, 224, 480, 544]
    ws = [W_out[offs[k]:offs[k + 1]] for k in range(5)]
    return _make_matmul(B)(*feats, *ws, b_out.reshape(1, 4))
